# baseline (speedup 1.0000x reference)
import functools

import jax
import jax.numpy as jnp
from jax import lax
from jax.experimental import pallas as pl
from jax.experimental.pallas import tpu as pltpu
from jax.experimental.pallas import tpu_sc as plsc

N_NODES = 10000
N_EDGES = 320000
D = 128

NC = 2
NS = 16
NW = NC * NS
EPT = N_EDGES // NW
C = 80
NCH = EPT // C
G = 25
NG = NCH // G
RPT = 624
VL = 16

_mesh = plsc.VectorSubcoreMesh(core_axis_name="c", subcore_axis_name="s")


@functools.partial(
    pl.kernel,
    mesh=_mesh,
    out_type=jax.ShapeDtypeStruct((NC, N_NODES, D), jnp.float32),
    scratch_types=[
        pltpu.VMEM((G, C), jnp.int32),
        pltpu.VMEM((G, C), jnp.int32),
        pltpu.VMEM((2, C, D), jnp.float32),
        pltpu.VMEM((2, C // 2, D), jnp.float32),
        pltpu.VMEM_SHARED((N_NODES, D), jnp.float32),
        pltpu.SemaphoreType.DMA,
        pltpu.SemaphoreType.DMA,
        pltpu.SemaphoreType.DMA,
        pltpu.SemaphoreType.DMA,
        pltpu.SemaphoreType.DMA,
        pltpu.SemaphoreType.DMA,
    ],
)
def _gine_scatter(nodes_hbm, src_hbm, dst_hbm, edges_hbm, out_hbm,
                  src_v, dst_v, rows_v, edg_v, acc,
                  gsem0, gsem1, esem0, esem1, ssem0, ssem1):
    cid = lax.axis_index("c")
    sid = lax.axis_index("s")
    wid = sid * NC + cid
    gsem = (gsem0, gsem1)
    esem = (esem0, esem1)
    ssem = (ssem0, ssem1)
    CH = C // 2

    def _zero(i, _):
        for h in range(D // VL):
            rows_v[0, i, pl.ds(h * VL, VL)] = jnp.zeros((VL,), jnp.float32)
        return 0

    lax.fori_loop(0, C, _zero, 0)
    full = RPT // C
    rem = RPT - full * C
    tail = N_NODES - NS * RPT
    base = sid * RPT
    for k in range(full):
        pltpu.sync_copy(rows_v.at[0], acc.at[pl.ds(base + k * C, C)])
    if rem:
        pltpu.sync_copy(rows_v.at[0, pl.ds(0, rem)],
                        acc.at[pl.ds(base + full * C, rem)])

    @pl.when(sid == NS - 1)
    def _zero_tail():
        pltpu.sync_copy(rows_v.at[0, pl.ds(0, tail)],
                        acc.at[pl.ds(NS * RPT, tail)])

    plsc.subcore_barrier()

    ebase = wid * EPT

    def _gather_desc(g, j, b):
        return pltpu.make_async_copy(
            nodes_hbm.at[src_v.at[j]], rows_v.at[b], gsem[b])

    def _edge_desc(g, j, h):
        return pltpu.make_async_copy(
            edges_hbm.at[pl.ds(ebase + (g * G + j) * C + h * CH, CH)],
            edg_v.at[h], esem[h])

    def _scat_desc(j, b):
        return pltpu.make_async_copy(rows_v.at[b], acc.at[dst_v.at[j]],
                                     ssem[b])

    def _compute(b, h):
        def _body(i, _):
            for k in range(D // VL):
                s = pl.ds(k * VL, VL)
                rows_v[b, h * CH + i, s] = jnp.maximum(
                    rows_v[b, h * CH + i, s] + edg_v[h, i, s], 0.0)
            return 0

        lax.fori_loop(0, CH, _body, 0)

    def _group(g, _):
        pltpu.sync_copy(src_hbm.at[wid, g], src_v)
        pltpu.sync_copy(dst_hbm.at[wid, g], dst_v)
        _gather_desc(g, 0, 0).start()
        _edge_desc(g, 0, 0).start()
        _edge_desc(g, 0, 1).start()

        def _pair(i, _):
            j0 = 2 * i
            _gather_desc(g, j0, 0).wait()
            _edge_desc(g, j0, 0).wait()
            _compute(0, 0)
            _edge_desc(g, j0 + 1, 0).start()

            @pl.when(i > 0)
            def _():
                _scat_desc(j0 - 1, 1).wait()

            _gather_desc(g, j0 + 1, 1).start()
            _edge_desc(g, j0, 1).wait()
            _compute(0, 1)
            _edge_desc(g, j0 + 1, 1).start()
            _scat_desc(j0, 0).start(add=True)
            _gather_desc(g, j0 + 1, 1).wait()
            _edge_desc(g, j0 + 1, 0).wait()
            _compute(1, 0)

            @pl.when(i < G // 2 - 1)
            def _():
                _edge_desc(g, j0 + 2, 0).start()

            _scat_desc(j0, 0).wait()

            @pl.when(i < G // 2 - 1)
            def _():
                _gather_desc(g, j0 + 2, 0).start()

            _edge_desc(g, j0 + 1, 1).wait()
            _compute(1, 1)

            @pl.when(i < G // 2 - 1)
            def _():
                _edge_desc(g, j0 + 2, 1).start()

            _scat_desc(j0 + 1, 1).start(add=True)
            return 0

        lax.fori_loop(0, G // 2, _pair, 0)
        _gather_desc(g, G - 1, 0).start()
        _edge_desc(g, G - 1, 0).start()
        _edge_desc(g, G - 1, 1).start()
        _scat_desc(G - 2, 1).wait()
        _gather_desc(g, G - 1, 0).wait()
        _edge_desc(g, G - 1, 0).wait()
        _compute(0, 0)
        _edge_desc(g, G - 1, 1).wait()
        _compute(0, 1)
        _scat_desc(G - 1, 0).start(add=True)
        _scat_desc(G - 1, 0).wait()
        return 0

    lax.fori_loop(0, NG, _group, 0)

    plsc.subcore_barrier()
    pltpu.sync_copy(acc.at[pl.ds(base, RPT)], out_hbm.at[cid, pl.ds(base, RPT)])

    @pl.when(sid == NS - 1)
    def _write_tail():
        pltpu.sync_copy(acc.at[pl.ds(NS * RPT, tail)],
                        out_hbm.at[cid, pl.ds(NS * RPT, tail)])


def _combine_body(scale_ref, n_ref, a_ref, b_ref, o_ref):
    o_ref[...] = scale_ref[0] * n_ref[...] + a_ref[0] + b_ref[0]


_GRID = 10
_combine = pl.pallas_call(
    _combine_body,
    grid=(_GRID,),
    in_specs=[
        pl.BlockSpec(memory_space=pltpu.SMEM),
        pl.BlockSpec((N_NODES // _GRID, D), lambda i: (i, 0)),
        pl.BlockSpec((1, N_NODES // _GRID, D), lambda i: (0, i, 0)),
        pl.BlockSpec((1, N_NODES // _GRID, D), lambda i: (1, i, 0)),
    ],
    out_specs=pl.BlockSpec((N_NODES // _GRID, D), lambda i: (i, 0)),
    out_shape=jax.ShapeDtypeStruct((N_NODES, D), jnp.float32),
)


def kernel(nodes, edge_index, edges, eps):
    ei = edge_index.astype(jnp.int32)
    src = ei[1].reshape(NW, NG, G, C)
    dst = ei[0].reshape(NW, NG, G, C)
    partials = _gine_scatter(nodes, src, dst, edges)
    scale = (1.0 + eps).astype(jnp.float32).reshape(1)
    return _combine(scale, nodes, partials, partials)


# speedup vs baseline: 1.1899x; 1.1899x over previous
import functools

import jax
import jax.numpy as jnp
from jax import lax
from jax.experimental import pallas as pl
from jax.experimental.pallas import tpu as pltpu
from jax.experimental.pallas import tpu_sc as plsc

N_NODES = 10000
N_EDGES = 320000
D = 128

NC = 2
NS = 16
NW = NC * NS
EPT = N_EDGES // NW
C = 80
NCH = EPT // C
G = 25
NG = NCH // G
RPT = 624
VL = 16

_mesh = plsc.VectorSubcoreMesh(core_axis_name="c", subcore_axis_name="s")


@functools.partial(
    pl.kernel,
    mesh=_mesh,
    out_type=jax.ShapeDtypeStruct((NC, N_NODES, D), jnp.float32),
    scratch_types=[
        pltpu.VMEM((G, C), jnp.int32),
        pltpu.VMEM((G, C), jnp.int32),
        pltpu.VMEM((2, C, D), jnp.float32),
        pltpu.VMEM((2, C // 2, D), jnp.float32),
        pltpu.VMEM_SHARED((N_NODES, D), jnp.float32),
        pltpu.SemaphoreType.DMA,
        pltpu.SemaphoreType.DMA,
        pltpu.SemaphoreType.DMA,
        pltpu.SemaphoreType.DMA,
        pltpu.SemaphoreType.DMA,
        pltpu.SemaphoreType.DMA,
    ],
)
def _gine_scatter(nodes_hbm, src_hbm, dst_hbm, edges_hbm, out_hbm,
                  src_v, dst_v, rows_v, edg_v, acc,
                  gsem0, gsem1, esem0, esem1, ssem0, ssem1):
    cid = lax.axis_index("c")
    sid = lax.axis_index("s")
    wid = sid * NC + cid
    gsem = (gsem0, gsem1)
    esem = (esem0, esem1)
    ssem = (ssem0, ssem1)
    CH = C // 2

    def _zero(i, _):
        for h in range(D // VL):
            rows_v[0, i, pl.ds(h * VL, VL)] = jnp.zeros((VL,), jnp.float32)
        return 0

    lax.fori_loop(0, C, _zero, 0)
    full = RPT // C
    rem = RPT - full * C
    tail = N_NODES - NS * RPT
    base = sid * RPT
    for k in range(full):
        pltpu.sync_copy(rows_v.at[0], acc.at[pl.ds(base + k * C, C)])
    if rem:
        pltpu.sync_copy(rows_v.at[0, pl.ds(0, rem)],
                        acc.at[pl.ds(base + full * C, rem)])

    @pl.when(sid == NS - 1)
    def _zero_tail():
        pltpu.sync_copy(rows_v.at[0, pl.ds(0, tail)],
                        acc.at[pl.ds(NS * RPT, tail)])

    plsc.subcore_barrier()

    ebase = wid * EPT

    def _gather_desc(g, j, b):
        return pltpu.make_async_copy(
            nodes_hbm.at[src_v.at[j]], rows_v.at[b], gsem[b])

    def _edge_desc(g, j, h):
        return pltpu.make_async_copy(
            edges_hbm.at[pl.ds(ebase + (g * G + j) * C + h * CH, CH)],
            edg_v.at[h], esem[h])

    def _scat_desc(j, b):
        return pltpu.make_async_copy(rows_v.at[b], acc.at[dst_v.at[j]],
                                     ssem[b])

    def _compute(b, h):
        def _body(i, _):
            for k in range(D // VL):
                s = pl.ds(k * VL, VL)
                rows_v[b, h * CH + i, s] = jnp.maximum(
                    rows_v[b, h * CH + i, s] + edg_v[h, i, s], 0.0)
            return 0

        lax.fori_loop(0, CH, _body, 0)

    def _group(g, _):
        pltpu.sync_copy(src_hbm.at[wid, g], src_v)
        pltpu.sync_copy(dst_hbm.at[wid, g], dst_v)
        _gather_desc(g, 0, 0).start()
        _edge_desc(g, 0, 0).start()
        _edge_desc(g, 0, 1).start()

        def _pair(i, _):
            j0 = 2 * i
            _gather_desc(g, j0, 0).wait()

            @pl.when(i > 0)
            def _():
                _scat_desc(j0 - 1, 1).wait()

            _gather_desc(g, j0 + 1, 1).start()
            _edge_desc(g, j0, 0).wait()
            _compute(0, 0)
            _edge_desc(g, j0 + 1, 0).start()
            _edge_desc(g, j0, 1).wait()
            _compute(0, 1)
            _edge_desc(g, j0 + 1, 1).start()
            _scat_desc(j0, 0).start(add=True)
            _gather_desc(g, j0 + 1, 1).wait()
            _scat_desc(j0, 0).wait()

            @pl.when(i < G // 2 - 1)
            def _():
                _gather_desc(g, j0 + 2, 0).start()

            _edge_desc(g, j0 + 1, 0).wait()
            _compute(1, 0)

            @pl.when(i < G // 2 - 1)
            def _():
                _edge_desc(g, j0 + 2, 0).start()

            _edge_desc(g, j0 + 1, 1).wait()
            _compute(1, 1)

            @pl.when(i < G // 2 - 1)
            def _():
                _edge_desc(g, j0 + 2, 1).start()

            _scat_desc(j0 + 1, 1).start(add=True)
            return 0

        lax.fori_loop(0, G // 2, _pair, 0)
        _gather_desc(g, G - 1, 0).start()
        _edge_desc(g, G - 1, 0).start()
        _edge_desc(g, G - 1, 1).start()
        _scat_desc(G - 2, 1).wait()
        _gather_desc(g, G - 1, 0).wait()
        _edge_desc(g, G - 1, 0).wait()
        _compute(0, 0)
        _edge_desc(g, G - 1, 1).wait()
        _compute(0, 1)
        _scat_desc(G - 1, 0).start(add=True)
        _scat_desc(G - 1, 0).wait()
        return 0

    lax.fori_loop(0, NG, _group, 0)

    plsc.subcore_barrier()
    pltpu.sync_copy(acc.at[pl.ds(base, RPT)], out_hbm.at[cid, pl.ds(base, RPT)])

    @pl.when(sid == NS - 1)
    def _write_tail():
        pltpu.sync_copy(acc.at[pl.ds(NS * RPT, tail)],
                        out_hbm.at[cid, pl.ds(NS * RPT, tail)])


def _combine_body(scale_ref, n_ref, a_ref, b_ref, o_ref):
    o_ref[...] = scale_ref[0] * n_ref[...] + a_ref[0] + b_ref[0]


_GRID = 10
_combine = pl.pallas_call(
    _combine_body,
    grid=(_GRID,),
    in_specs=[
        pl.BlockSpec(memory_space=pltpu.SMEM),
        pl.BlockSpec((N_NODES // _GRID, D), lambda i: (i, 0)),
        pl.BlockSpec((1, N_NODES // _GRID, D), lambda i: (0, i, 0)),
        pl.BlockSpec((1, N_NODES // _GRID, D), lambda i: (1, i, 0)),
    ],
    out_specs=pl.BlockSpec((N_NODES // _GRID, D), lambda i: (i, 0)),
    out_shape=jax.ShapeDtypeStruct((N_NODES, D), jnp.float32),
)


def kernel(nodes, edge_index, edges, eps):
    ei = edge_index.astype(jnp.int32)
    src = ei[1].reshape(NW, NG, G, C)
    dst = ei[0].reshape(NW, NG, G, C)
    partials = _gine_scatter(nodes, src, dst, edges)
    scale = (1.0 + eps).astype(jnp.float32).reshape(1)
    return _combine(scale, nodes, partials, partials)
